# Initial kernel scaffold; baseline (speedup 1.0000x reference)
import jax
import jax.numpy as jnp
from jax.experimental import pallas as pl


def kernel(x, adj_entity, adj_relation):
    raise NotImplementedError("write your pallas kernel here")


# speedup vs baseline: 10.3317x; 10.3317x over previous
import functools

import jax
import jax.numpy as jnp
from jax import lax
from jax.experimental import pallas as pl
from jax.experimental.pallas import tpu as pltpu
from jax.experimental.pallas import tpu_sc as plsc

N_ENTITY = 100000
N_NEIGHBOR = 16
BATCH = 16384

NC = 2
NS = 16
NW = NC * NS
SPW = BATCH // NW
NIDX = SPW * N_NEIGHBOR
K = 1024
NCHUNK = NIDX // K


ROWS_PER_CHUNK = K // N_NEIGHBOR


def _rf_body(x_hbm, ent_hbm, rel_hbm,
             out1, out2, out3, out4,
             idx0_v, ent1_v, rel1_v, idx1_v, ent2_v, rel2_v,
             sem_h1e, sem_h1r, sem_e0, sem_e1, sem_r0, sem_r1):
    wid = lax.axis_index("s") * NC + lax.axis_index("c")
    base = wid * SPW

    pltpu.sync_copy(x_hbm.at[pl.ds(base, SPW)], idx0_v)

    cp_e1 = pltpu.async_copy(ent_hbm.at[idx0_v], ent1_v, sem_h1e)
    cp_r1 = pltpu.async_copy(rel_hbm.at[idx0_v], rel1_v, sem_h1r)
    cp_e1.wait()

    def repack(c):
        for i in range(ROWS_PER_CHUNK):
            r = c * ROWS_PER_CHUNK + i
            idx1_v[pl.ds(r * N_NEIGHBOR, N_NEIGHBOR)] = ent1_v[r]

    sem_e = (sem_e0, sem_e1)
    sem_r = (sem_r0, sem_r1)
    cp_e = [None, None]
    cp_r = [None, None]
    obase = base * N_NEIGHBOR

    for c in range(NCHUNK + 1):
        if c < NCHUNK:
            b = c % 2
            repack(c)
            idx_c = idx1_v.at[pl.ds(c * K, K)]
            cp_e[b] = pltpu.async_copy(ent_hbm.at[idx_c], ent2_v.at[b], sem_e[b])
            cp_r[b] = pltpu.async_copy(rel_hbm.at[idx_c], rel2_v.at[b], sem_r[b])
        if c == 0:
            sync = pltpu.sync_copy
            sync(ent1_v, out1.at[pl.ds(base, SPW)])
            cp_r1.wait()
            sync(rel1_v, out3.at[pl.ds(base, SPW)])
        else:
            pb = (c - 1) % 2
            row = obase + (c - 1) * K
            cp_e[pb].wait()
            pltpu.sync_copy(ent2_v.at[pb], out2.at[pl.ds(row, K)])
            cp_r[pb].wait()
            pltpu.sync_copy(rel2_v.at[pb], out4.at[pl.ds(row, K)])


@jax.jit
def kernel(x, adj_entity, adj_relation):
    x_flat = x.reshape(BATCH).astype(jnp.int32)
    ent = adj_entity.astype(jnp.int32)
    rel = adj_relation.astype(jnp.int32)

    i32 = jnp.int32
    run = pl.kernel(
        _rf_body,
        out_type=(
            jax.ShapeDtypeStruct((BATCH, N_NEIGHBOR), i32),
            jax.ShapeDtypeStruct((BATCH * N_NEIGHBOR, N_NEIGHBOR), i32),
            jax.ShapeDtypeStruct((BATCH, N_NEIGHBOR), i32),
            jax.ShapeDtypeStruct((BATCH * N_NEIGHBOR, N_NEIGHBOR), i32),
        ),
        mesh=plsc.VectorSubcoreMesh(core_axis_name="c", subcore_axis_name="s"),
        compiler_params=pltpu.CompilerParams(use_tc_tiling_on_sc=False),
        scratch_types=[
            pltpu.VMEM((SPW,), i32),
            pltpu.VMEM((SPW, N_NEIGHBOR), i32),
            pltpu.VMEM((SPW, N_NEIGHBOR), i32),
            pltpu.VMEM((NIDX,), i32),
            pltpu.VMEM((2, K, N_NEIGHBOR), i32),
            pltpu.VMEM((2, K, N_NEIGHBOR), i32),
            pltpu.SemaphoreType.DMA,
            pltpu.SemaphoreType.DMA,
            pltpu.SemaphoreType.DMA,
            pltpu.SemaphoreType.DMA,
            pltpu.SemaphoreType.DMA,
            pltpu.SemaphoreType.DMA,
        ],
    )
    ent1, ent2, rel1, rel2 = run(x_flat, ent, rel)
    return (
        x,
        ent1,
        ent2.reshape(BATCH, N_NEIGHBOR * N_NEIGHBOR),
        rel1,
        rel2.reshape(BATCH, N_NEIGHBOR * N_NEIGHBOR),
    )
